# baseline (speedup 1.0000x reference)
import functools

import jax
import jax.numpy as jnp
from jax import lax
from jax.experimental import pallas as pl
from jax.experimental.pallas import tpu as pltpu
from jax.experimental.pallas import tpu_sc as plsc

NUM_EMBEDDINGS = 1000000
EMBEDDING_DIM = 64
BATCH = 4096
SEQ = 200
TOTAL = BATCH * SEQ

_info = plsc.get_sparse_core_info()
NC, NS = _info.num_cores, _info.num_subcores
NW = NC * NS
PER_W = TOTAL // NW
CHUNK = 128
NCHUNK = PER_W // CHUNK


def _make_gather():
    mesh = plsc.VectorSubcoreMesh(core_axis_name="c", subcore_axis_name="s")

    @functools.partial(
        pl.kernel,
        mesh=mesh,
        compiler_params=pltpu.CompilerParams(use_tc_tiling_on_sc=False),
        out_type=jax.ShapeDtypeStruct((TOTAL, EMBEDDING_DIM), jnp.float32),
        scratch_types=[
            pltpu.VMEM((CHUNK,), jnp.int32),
            pltpu.VMEM((CHUNK, EMBEDDING_DIM), jnp.float32),
            pltpu.SemaphoreType.DMA,
        ],
    )
    def gather_kernel(idx_hbm, table_hbm, out_hbm, idx_v, rows_v, sem):
        wid = lax.axis_index("s") * NC + lax.axis_index("c")
        wbase = wid * PER_W

        def body(c, _):
            base = wbase + c * CHUNK
            pltpu.sync_copy(idx_hbm.at[pl.ds(base, CHUNK)], idx_v)
            pltpu.async_copy(table_hbm.at[idx_v], rows_v, sem).wait()
            pltpu.sync_copy(rows_v, out_hbm.at[pl.ds(base, CHUNK)])
            return _

        lax.fori_loop(0, NCHUNK, body, 0, unroll=False)

    return gather_kernel


_gather = _make_gather()


@jax.jit
def kernel(input_ids, weight):
    flat_ids = input_ids.reshape(TOTAL)
    out = _gather(flat_ids, weight)
    return out.reshape(BATCH, SEQ, EMBEDDING_DIM)


# speedup vs baseline: 1.1922x; 1.1922x over previous
import functools

import jax
import jax.numpy as jnp
from jax import lax
from jax.experimental import pallas as pl
from jax.experimental.pallas import tpu as pltpu
from jax.experimental.pallas import tpu_sc as plsc

NUM_EMBEDDINGS = 1000000
EMBEDDING_DIM = 64
BATCH = 4096
SEQ = 200

_info = plsc.get_sparse_core_info()
NC, NS = _info.num_cores, _info.num_subcores
NW = NC * NS
ROWS_W = BATCH // NW
C0, C1 = 128, SEQ - 128
NBUF = 8
GROUP = NBUF // 2
NSUPER = ROWS_W // GROUP


def _make_gather():
    mesh = plsc.VectorSubcoreMesh(core_axis_name="c", subcore_axis_name="s")

    @functools.partial(
        pl.kernel,
        mesh=mesh,
        compiler_params=pltpu.CompilerParams(use_tc_tiling_on_sc=False),
        out_type=jax.ShapeDtypeStruct((BATCH, SEQ, EMBEDDING_DIM), jnp.float32),
        scratch_types=[
            pltpu.VMEM((ROWS_W, SEQ), jnp.int32),
            pltpu.VMEM((NBUF, C0, EMBEDDING_DIM), jnp.float32),
        ]
        + [pltpu.SemaphoreType.DMA] * (2 * NBUF),
    )
    def gather_kernel(idx_hbm, table_hbm, out_hbm, idx_v, rows_v, *sems):
        g_sem = sems[:NBUF]
        w_sem = sems[NBUF:]
        wid = lax.axis_index("s") * NC + lax.axis_index("c")
        rbase = wid * ROWS_W

        pltpu.sync_copy(idx_hbm.at[pl.ds(rbase, ROWS_W)], idx_v)

        def super_step(s, _):
            @pl.when(s > 0)
            def _drain():
                for j in range(GROUP):
                    for h, off, sz in ((0, 0, C0), (1, C0, C1)):
                        b = 2 * j + h
                        row = rbase + s * GROUP + j
                        pltpu.make_async_copy(
                            rows_v.at[b, pl.ds(0, sz)],
                            out_hbm.at[row, pl.ds(off, sz)],
                            w_sem[b],
                        ).wait()

            handles = []
            for j in range(GROUP):
                r = s * GROUP + j
                for h, off, sz in ((0, 0, C0), (1, C0, C1)):
                    b = 2 * j + h
                    handles.append(
                        pltpu.async_copy(
                            table_hbm.at[idx_v.at[r, pl.ds(off, sz)]],
                            rows_v.at[b, pl.ds(0, sz)],
                            g_sem[b],
                        )
                    )
            k = 0
            for j in range(GROUP):
                row = rbase + s * GROUP + j
                for h, off, sz in ((0, 0, C0), (1, C0, C1)):
                    b = 2 * j + h
                    handles[k].wait()
                    k += 1
                    pltpu.async_copy(
                        rows_v.at[b, pl.ds(0, sz)],
                        out_hbm.at[row, pl.ds(off, sz)],
                        w_sem[b],
                    )
            return _

        lax.fori_loop(0, NSUPER, super_step, 0, unroll=False)

        for j in range(GROUP):
            for h, off, sz in ((0, 0, C0), (1, C0, C1)):
                b = 2 * j + h
                row = rbase + (NSUPER - 1) * GROUP + j
                pltpu.make_async_copy(
                    rows_v.at[b, pl.ds(0, sz)],
                    out_hbm.at[row, pl.ds(off, sz)],
                    w_sem[b],
                ).wait()

    return gather_kernel


_gather = _make_gather()


@jax.jit
def kernel(input_ids, weight):
    return _gather(input_ids, weight)
